# baseline (speedup 1.0000x reference)
import functools

import jax
import jax.numpy as jnp
from jax import lax
from jax.experimental import pallas as pl
from jax.experimental.pallas import tpu as pltpu
from jax.experimental.pallas import tpu_sc as plsc

B, L, N, C, M = 2, 10, 8192, 100, 32
NCORES, NSUB = 2, 16
NW = NCORES * NSUB
P = N // NW
NCOMBO = 2 * B
CP = 128
ACC = L * M * CP
ROWS = NCOMBO * L * M
CTAIL = 84


def _sc_kernel_body(pred0, pred1, masks0, masks1, zeros_acc, zeros_cnt,
                    sums_out, cnt_out,
                    mbuf, idx_ref, pbuf, acc, cnt):
    cid = lax.axis_index("c")
    sid = lax.axis_index("s")
    wid = sid * NCORES + cid
    p0 = wid * P

    pltpu.sync_copy(zeros_cnt, cnt)

    iota = jax.lax.broadcasted_iota(jnp.int32, (16,), 0)
    tail_keep = iota >= (2 * 16 - (C - CTAIL))
    ones16 = jnp.ones((16,), jnp.float32)

    masks = (masks0, masks1)
    for combo in range(NCOMBO):
        b, f = combo // 2, combo % 2
        pltpu.sync_copy(masks[f].at[b, :, pl.ds(p0, P)], mbuf)

        def _group(g, _):
            col = g * 16
            best = mbuf[0, pl.ds(col, 16)]
            bidx = jnp.zeros((16,), jnp.int32)

            def _scan_m(m, carry):
                best, bidx = carry
                v = mbuf[m, pl.ds(col, 16)]
                gt = v > best
                bidx = jnp.where(gt, jnp.full((16,), 1, jnp.int32) * m, bidx)
                best = jnp.maximum(v, best)
                return best, bidx

            _, bidx = lax.fori_loop(1, M, _scan_m, (best, bidx))
            idx_ref[combo * 2 + g // 8, pl.ds((g % 8) * 16, 16)] = bidx
            return 0

        lax.fori_loop(0, P // 16, _group, 0)

    preds = (pred0, pred1)
    for combo in range(NCOMBO):
        b, f = combo // 2, combo % 2

        def _grp_cnt(g, _):
            bidx = idx_ref[combo * 2 + g // 8, pl.ds((g % 8) * 16, 16)]
            for j in range(16):
                m = bidx[j]
                plsc.addupdate(cnt.at[pl.ds((combo * M + m) * CP, 16)], ones16)
            return 0

        lax.fori_loop(0, P // 16, _grp_cnt, 0)

        pltpu.sync_copy(zeros_acc, acc)

        def _layer(l, _):
            pltpu.sync_copy(
                preds[f].at[pl.ds((b * L + l) * N * C + p0 * C, P * C)], pbuf)

            def _grp(g, _):
                bidx = idx_ref[combo * 2 + g // 8, pl.ds((g % 8) * 16, 16)]
                for j in range(16):
                    m = bidx[j]
                    ab = (l * M + m) * CP
                    pb = (g * 16 + j) * C
                    for k in range(C // 16):
                        v = pbuf[pl.ds(pb + k * 16, 16)]
                        plsc.addupdate(acc.at[pl.ds(ab + k * 16, 16)], v)
                    v = pbuf[pl.ds(pb + CTAIL, 16)]
                    v = jnp.where(tail_keep, v, 0.0)
                    plsc.addupdate(acc.at[pl.ds(ab + CTAIL, 16)], v)
                return 0

            lax.fori_loop(0, P // 16, _grp, 0)
            return 0

        lax.fori_loop(0, L, _layer, 0)

        pltpu.sync_copy(acc, sums_out.at[wid, pl.ds(combo * ACC, ACC)])

    pltpu.sync_copy(cnt, cnt_out.at[wid])


def _make_sc_kernel():
    mesh = plsc.VectorSubcoreMesh(core_axis_name="c", subcore_axis_name="s")
    return pl.kernel(
        _sc_kernel_body,
        out_type=[
            jax.ShapeDtypeStruct((NW, NCOMBO * ACC), jnp.float32),
            jax.ShapeDtypeStruct((NW, NCOMBO * M * CP), jnp.float32),
        ],
        mesh=mesh,
        scratch_types=[
            pltpu.VMEM((M, P), jnp.float32),
            pltpu.VMEM((NCOMBO * 2, 128), jnp.int32),
            pltpu.VMEM((P * C,), jnp.float32),
            pltpu.VMEM((ACC,), jnp.float32),
            pltpu.VMEM((NCOMBO * M * CP,), jnp.float32),
        ],
    )


def _tc_body(s_ref, c_ref, o_ref):
    S = jnp.sum(s_ref[...], axis=0)
    K = jnp.sum(c_ref[...], axis=0)
    S4 = S.reshape(NCOMBO, L, M, CP)[..., :C]
    cnt = K.reshape(NCOMBO, M, CP)[:, :, 0:1].reshape(NCOMBO, 1, M, 1)
    denom = jnp.maximum(cnt, 1.0)
    fmap = jnp.where(cnt > 0, S4 / denom, 0.0)

    loss = jnp.zeros((L,), jnp.float32)
    nobj = jnp.zeros((), jnp.float32)
    for b in range(B):
        f1 = fmap[2 * b + 0]
        f2 = fmap[2 * b + 1]
        mask_obj = jnp.logical_and(jnp.sum(f1[0], axis=1) != 0,
                                   jnp.sum(f2[0], axis=1) != 0)
        maskf = mask_obj.astype(jnp.float32)
        t1 = f1 - jnp.max(f1, axis=2, keepdims=True)
        tgt = jnp.exp(t1)
        tgt = tgt / jnp.sum(tgt, axis=2, keepdims=True)
        t2 = f2 - jnp.max(f2, axis=2, keepdims=True)
        logp = t2 - jnp.log(jnp.sum(jnp.exp(t2), axis=2, keepdims=True))
        CE = -jnp.sum(tgt * logp, axis=2)
        loss = loss + jnp.sum(CE * maskf[None, :], axis=1) / jnp.maximum(
            jnp.sum(maskf), 1.0)
        nobj = nobj + jnp.sum(maskf)
    o_ref[...] = loss / jnp.maximum(nobj, 1.0)


def _tc_tail(sums, cnts):
    return pl.pallas_call(
        _tc_body,
        out_shape=jax.ShapeDtypeStruct((L,), jnp.float32),
    )(sums, cnts)


@jax.jit
def kernel(pred0, pred1, masks0, masks1):
    zeros_acc = jnp.zeros((ACC,), jnp.float32)
    zeros_cnt = jnp.zeros((NCOMBO * M * CP,), jnp.float32)
    sums, cnts = _make_sc_kernel()(pred0.reshape(-1), pred1.reshape(-1),
                                   masks0, masks1, zeros_acc, zeros_cnt)
    return _tc_tail(sums, cnts)


# speedup vs baseline: 1.1770x; 1.1770x over previous
import functools

import jax
import jax.numpy as jnp
from jax import lax
from jax.experimental import pallas as pl
from jax.experimental.pallas import tpu as pltpu
from jax.experimental.pallas import tpu_sc as plsc

B, L, N, C, M = 2, 10, 8192, 100, 32
NCORES, NSUB = 2, 16
NW = NCORES * NSUB
P = N // NW
NCOMBO = 2 * B
CP = 128
ACC = L * M * CP
ROWS = NCOMBO * L * M
CTAIL = 84


def _sc_kernel_body(pred0, pred1, masks0, masks1, zeros_acc, zeros_cnt,
                    sums_out, cnt_out,
                    mbuf, idx_ref, pbuf, acc, cnt):
    cid = lax.axis_index("c")
    sid = lax.axis_index("s")
    wid = sid * NCORES + cid
    p0 = wid * P

    pltpu.sync_copy(zeros_cnt, cnt)

    iota = jax.lax.broadcasted_iota(jnp.int32, (16,), 0)
    tail_keep = iota >= (2 * 16 - (C - CTAIL))
    ones16 = jnp.ones((16,), jnp.float32)

    masks = (masks0, masks1)
    for combo in range(NCOMBO):
        b, f = combo // 2, combo % 2
        pltpu.sync_copy(masks[f].at[b, :, pl.ds(p0, P)], mbuf)

        def _group(g, _):
            col = g * 16
            best = mbuf[0, pl.ds(col, 16)]
            bidx = jnp.zeros((16,), jnp.int32)

            def _scan_m(m, carry):
                best, bidx = carry
                v = mbuf[m, pl.ds(col, 16)]
                gt = v > best
                bidx = jnp.where(gt, jnp.full((16,), 1, jnp.int32) * m, bidx)
                best = jnp.maximum(v, best)
                return best, bidx

            _, bidx = lax.fori_loop(1, M, _scan_m, (best, bidx))
            idx_ref[combo * 2 + g // 8, pl.ds((g % 8) * 16, 16)] = bidx
            return 0

        lax.fori_loop(0, P // 16, _group, 0)

    preds = (pred0, pred1)
    for combo in range(NCOMBO):
        b, f = combo // 2, combo % 2

        def _grp_cnt(g, _):
            bidx = idx_ref[combo * 2 + g // 8, pl.ds((g % 8) * 16, 16)]
            for j in range(16):
                m = bidx[j]
                plsc.addupdate(cnt.at[pl.ds((combo * M + m) * CP, 16)], ones16)
            return 0

        lax.fori_loop(0, P // 16, _grp_cnt, 0)

        pltpu.sync_copy(zeros_acc, acc)

        def _layer(l, _):
            pltpu.sync_copy(preds[f].at[b, l, pl.ds(p0, P), :], pbuf)

            def _grp(g, _):
                bidx = idx_ref[combo * 2 + g // 8, pl.ds((g % 8) * 16, 16)]
                for j in range(16):
                    m = bidx[j]
                    ab = (l * M + m) * CP
                    pr = g * 16 + j
                    for k in range(C // 16):
                        v = pbuf[pr, pl.ds(k * 16, 16)]
                        plsc.addupdate(acc.at[pl.ds(ab + k * 16, 16)], v)
                    v = pbuf[pr, pl.ds(CTAIL, 16)]
                    v = jnp.where(tail_keep, v, 0.0)
                    plsc.addupdate(acc.at[pl.ds(ab + CTAIL, 16)], v)
                return 0

            lax.fori_loop(0, P // 16, _grp, 0)
            return 0

        lax.fori_loop(0, L, _layer, 0)

        pltpu.sync_copy(acc, sums_out.at[wid, pl.ds(combo * ACC, ACC)])

    pltpu.sync_copy(cnt, cnt_out.at[wid])


def _make_sc_kernel():
    mesh = plsc.VectorSubcoreMesh(core_axis_name="c", subcore_axis_name="s")
    return pl.kernel(
        _sc_kernel_body,
        out_type=[
            jax.ShapeDtypeStruct((NW, NCOMBO * ACC), jnp.float32),
            jax.ShapeDtypeStruct((NW, NCOMBO * M * CP), jnp.float32),
        ],
        mesh=mesh,
        compiler_params=pltpu.CompilerParams(use_tc_tiling_on_sc=True),
        scratch_types=[
            pltpu.VMEM((M, P), jnp.float32),
            pltpu.VMEM((NCOMBO * 2, 128), jnp.int32),
            pltpu.VMEM((P, C), jnp.float32),
            pltpu.VMEM((ACC,), jnp.float32),
            pltpu.VMEM((NCOMBO * M * CP,), jnp.float32),
        ],
    )


def _tc_body(s_ref, c_ref, o_ref):
    S = jnp.sum(s_ref[...], axis=0)
    K = jnp.sum(c_ref[...], axis=0)
    S4 = S.reshape(NCOMBO, L, M, CP)[..., :C]
    cnt = K.reshape(NCOMBO, M, CP)[:, :, 0:1].reshape(NCOMBO, 1, M, 1)
    denom = jnp.maximum(cnt, 1.0)
    fmap = jnp.where(cnt > 0, S4 / denom, 0.0)

    loss = jnp.zeros((L,), jnp.float32)
    nobj = jnp.zeros((), jnp.float32)
    for b in range(B):
        f1 = fmap[2 * b + 0]
        f2 = fmap[2 * b + 1]
        mask_obj = jnp.logical_and(jnp.sum(f1[0], axis=1) != 0,
                                   jnp.sum(f2[0], axis=1) != 0)
        maskf = mask_obj.astype(jnp.float32)
        t1 = f1 - jnp.max(f1, axis=2, keepdims=True)
        tgt = jnp.exp(t1)
        tgt = tgt / jnp.sum(tgt, axis=2, keepdims=True)
        t2 = f2 - jnp.max(f2, axis=2, keepdims=True)
        logp = t2 - jnp.log(jnp.sum(jnp.exp(t2), axis=2, keepdims=True))
        CE = -jnp.sum(tgt * logp, axis=2)
        loss = loss + jnp.sum(CE * maskf[None, :], axis=1) / jnp.maximum(
            jnp.sum(maskf), 1.0)
        nobj = nobj + jnp.sum(maskf)
    o_ref[...] = loss / jnp.maximum(nobj, 1.0)


def _tc_tail(sums, cnts):
    return pl.pallas_call(
        _tc_body,
        out_shape=jax.ShapeDtypeStruct((L,), jnp.float32),
    )(sums, cnts)


@jax.jit
def kernel(pred0, pred1, masks0, masks1):
    zeros_acc = jnp.zeros((ACC,), jnp.float32)
    zeros_cnt = jnp.zeros((NCOMBO * M * CP,), jnp.float32)
    sums, cnts = _make_sc_kernel()(pred0, pred1,
                                   masks0, masks1, zeros_acc, zeros_cnt)
    return _tc_tail(sums, cnts)


# speedup vs baseline: 1.3445x; 1.1424x over previous
import functools

import jax
import jax.numpy as jnp
from jax import lax
from jax.experimental import pallas as pl
from jax.experimental.pallas import tpu as pltpu
from jax.experimental.pallas import tpu_sc as plsc

B, L, N, C, M = 2, 10, 8192, 100, 32
NCORES, NSUB = 2, 16
NW = NCORES * NSUB
P = N // NW
NCOMBO = 2 * B
CP = 128
ACC = L * M * CP
ROWS = NCOMBO * L * M
CTAIL = 84


def _sc_kernel_body(pred0, pred1, masks0, masks1, zeros_acc, zeros_cnt,
                    sums_out, cnt_out,
                    mbuf, idx_ref, pbuf0, pbuf1, acc, cnt, idxs,
                    sem0, sem1):
    cid = lax.axis_index("c")
    sid = lax.axis_index("s")
    wid = sid * NCORES + cid
    p0 = wid * P

    pltpu.sync_copy(zeros_cnt, cnt)

    iota = jax.lax.broadcasted_iota(jnp.int32, (16,), 0)
    tail_keep = iota >= (2 * 16 - (C - CTAIL))
    ones16 = jnp.ones((16,), jnp.float32)

    masks = (masks0, masks1)
    for combo in range(NCOMBO):
        b, f = combo // 2, combo % 2
        pltpu.sync_copy(masks[f].at[b, :, pl.ds(p0, P)], mbuf)

        def _group(g, _):
            col = g * 16
            best = mbuf[0, pl.ds(col, 16)]
            bidx = jnp.zeros((16,), jnp.int32)

            def _scan_m(m, carry):
                best, bidx = carry
                v = mbuf[m, pl.ds(col, 16)]
                gt = v > best
                bidx = jnp.where(gt, jnp.full((16,), 1, jnp.int32) * m, bidx)
                best = jnp.maximum(v, best)
                return best, bidx

            _, bidx = lax.fori_loop(1, M, _scan_m, (best, bidx))
            idx_ref[combo * 2 + g // 8, pl.ds((g % 8) * 16, 16)] = bidx
            return 0

        lax.fori_loop(0, P // 16, _group, 0)

    preds = (pred0, pred1)
    pbufs = (pbuf0, pbuf1)
    sems = (sem0, sem1)
    for combo in range(NCOMBO):
        b, f = combo // 2, combo % 2
        src = preds[f]

        def _grp_idx(g, _):
            bidx = idx_ref[combo * 2 + g // 8, pl.ds((g % 8) * 16, 16)]
            for j in range(16):
                m = bidx[j]
                idxs[g * 16 + j] = m * CP
                plsc.addupdate(cnt.at[pl.ds((combo * M + m) * CP, 16)], ones16)
            return 0

        lax.fori_loop(0, P // 16, _grp_idx, 0)

        pltpu.sync_copy(zeros_acc, acc)

        HP = P // 2
        pltpu.async_copy(src.at[b, 0, pl.ds(p0, HP), :], pbuf0, sem0)
        pltpu.async_copy(src.at[b, 0, pl.ds(p0 + HP, HP), :], pbuf1, sem1)

        def _layer(l, _):
            for d in range(2):
                pbuf = pbufs[d]
                sem = sems[d]
                pltpu.make_async_copy(src.at[b, 0, pl.ds(p0, HP), :],
                                      pbuf, sem).wait()

                def _grp(g, _, d=d, l=l, pbuf=pbuf):
                    gg = d * (HP // 16) + g
                    for j in range(16):
                        mo = idxs[gg * 16 + j]
                        ab = l * M * CP + mo
                        pr = g * 16 + j
                        for k in range(C // 16):
                            v = pbuf[pr, pl.ds(k * 16, 16)]
                            plsc.addupdate(acc.at[pl.ds(ab + k * 16, 16)], v)
                        v = pbuf[pr, pl.ds(CTAIL, 16)]
                        v = jnp.where(tail_keep, v, 0.0)
                        plsc.addupdate(acc.at[pl.ds(ab + CTAIL, 16)], v)
                    return 0

                lax.fori_loop(0, HP // 16, _grp, 0)

                @pl.when(l + 1 < L)
                def _next(d=d, l=l, pbuf=pbuf, sem=sem):
                    pltpu.async_copy(
                        src.at[b, l + 1, pl.ds(p0 + d * HP, HP), :],
                        pbuf, sem)
            return 0

        lax.fori_loop(0, L, _layer, 0)

        pltpu.sync_copy(acc, sums_out.at[wid, pl.ds(combo * ACC, ACC)])

    pltpu.sync_copy(cnt, cnt_out.at[wid])


def _make_sc_kernel():
    mesh = plsc.VectorSubcoreMesh(core_axis_name="c", subcore_axis_name="s")
    return pl.kernel(
        _sc_kernel_body,
        out_type=[
            jax.ShapeDtypeStruct((NW, NCOMBO * ACC), jnp.float32),
            jax.ShapeDtypeStruct((NW, NCOMBO * M * CP), jnp.float32),
        ],
        mesh=mesh,
        compiler_params=pltpu.CompilerParams(use_tc_tiling_on_sc=True),
        scratch_types=[
            pltpu.VMEM((M, P), jnp.float32),
            pltpu.VMEM((NCOMBO * 2, 128), jnp.int32),
            pltpu.VMEM((P // 2, C), jnp.float32),
            pltpu.VMEM((P // 2, C), jnp.float32),
            pltpu.VMEM((ACC,), jnp.float32),
            pltpu.VMEM((NCOMBO * M * CP,), jnp.float32),
            pltpu.SMEM((P,), jnp.int32),
            pltpu.SemaphoreType.DMA,
            pltpu.SemaphoreType.DMA,
        ],
    )


def _tc_body(s_ref, c_ref, o_ref):
    S = jnp.sum(s_ref[...], axis=0)
    K = jnp.sum(c_ref[...], axis=0)
    S4 = S.reshape(NCOMBO, L, M, CP)[..., :C]
    cnt = K.reshape(NCOMBO, M, CP)[:, :, 0:1].reshape(NCOMBO, 1, M, 1)
    denom = jnp.maximum(cnt, 1.0)
    fmap = jnp.where(cnt > 0, S4 / denom, 0.0)

    loss = jnp.zeros((L,), jnp.float32)
    nobj = jnp.zeros((), jnp.float32)
    for b in range(B):
        f1 = fmap[2 * b + 0]
        f2 = fmap[2 * b + 1]
        mask_obj = jnp.logical_and(jnp.sum(f1[0], axis=1) != 0,
                                   jnp.sum(f2[0], axis=1) != 0)
        maskf = mask_obj.astype(jnp.float32)
        t1 = f1 - jnp.max(f1, axis=2, keepdims=True)
        tgt = jnp.exp(t1)
        tgt = tgt / jnp.sum(tgt, axis=2, keepdims=True)
        t2 = f2 - jnp.max(f2, axis=2, keepdims=True)
        logp = t2 - jnp.log(jnp.sum(jnp.exp(t2), axis=2, keepdims=True))
        CE = -jnp.sum(tgt * logp, axis=2)
        loss = loss + jnp.sum(CE * maskf[None, :], axis=1) / jnp.maximum(
            jnp.sum(maskf), 1.0)
        nobj = nobj + jnp.sum(maskf)
    o_ref[...] = loss / jnp.maximum(nobj, 1.0)


def _tc_tail(sums, cnts):
    return pl.pallas_call(
        _tc_body,
        out_shape=jax.ShapeDtypeStruct((L,), jnp.float32),
    )(sums, cnts)


@jax.jit
def kernel(pred0, pred1, masks0, masks1):
    zeros_acc = jnp.zeros((ACC,), jnp.float32)
    zeros_cnt = jnp.zeros((NCOMBO * M * CP,), jnp.float32)
    sums, cnts = _make_sc_kernel()(pred0, pred1,
                                   masks0, masks1, zeros_acc, zeros_cnt)
    return _tc_tail(sums, cnts)


# speedup vs baseline: 1.6633x; 1.2371x over previous
import functools

import jax
import jax.numpy as jnp
from jax import lax
from jax.experimental import pallas as pl
from jax.experimental.pallas import tpu as pltpu
from jax.experimental.pallas import tpu_sc as plsc

B, L, N, C, M = 2, 10, 8192, 100, 32
NCORES, NSUB = 2, 16
NW = NCORES * NSUB
P = N // NW
HP = P // 2
CP = 128
ACC = L * M * CP
CTAIL = 84


def _sc_kernel_body(pred, masksf, zeros_acc, zeros_cnt,
                    sums_out, cnt_out,
                    mbuf, idx_ref, pbuf0, pbuf1, acc, cnt, idxs,
                    sem0, sem1):
    cid = lax.axis_index("c")
    sid = lax.axis_index("s")
    wid = sid * NCORES + cid
    p0 = wid * P

    pltpu.sync_copy(zeros_cnt, cnt)

    iota = jax.lax.broadcasted_iota(jnp.int32, (16,), 0)
    tail_keep = iota >= (2 * 16 - (C - CTAIL))
    ones16 = jnp.ones((16,), jnp.float32)

    for b in range(B):
        pltpu.sync_copy(masksf.at[b, :, pl.ds(p0, P)], mbuf)

        def _group(g, _, b=b):
            col = g * 16
            best = mbuf[0, pl.ds(col, 16)]
            bidx = jnp.zeros((16,), jnp.int32)

            def _scan_m(m, carry):
                best, bidx = carry
                v = mbuf[m, pl.ds(col, 16)]
                gt = v > best
                bidx = jnp.where(gt, jnp.full((16,), 1, jnp.int32) * m, bidx)
                best = jnp.maximum(v, best)
                return best, bidx

            _, bidx = lax.fori_loop(1, M, _scan_m, (best, bidx))
            idx_ref[b * 2 + g // 8, pl.ds((g % 8) * 16, 16)] = bidx
            return 0

        lax.fori_loop(0, P // 16, _group, 0)

    pbufs = (pbuf0, pbuf1)
    sems = (sem0, sem1)
    for b in range(B):
        def _grp_idx(g, _, b=b):
            bidx = idx_ref[b * 2 + g // 8, pl.ds((g % 8) * 16, 16)]
            for j in range(16):
                m = bidx[j]
                idxs[g * 16 + j] = m * CP
                plsc.addupdate(cnt.at[pl.ds((b * M + m) * CP, 16)], ones16)
            return 0

        lax.fori_loop(0, P // 16, _grp_idx, 0)

        pltpu.sync_copy(zeros_acc, acc)

        pltpu.async_copy(pred.at[b, 0, pl.ds(p0, HP), :], pbuf0, sem0)
        pltpu.async_copy(pred.at[b, 0, pl.ds(p0 + HP, HP), :], pbuf1, sem1)

        def _layer(l, _, b=b):
            for d in range(2):
                pbuf = pbufs[d]
                sem = sems[d]
                pltpu.make_async_copy(pred.at[b, 0, pl.ds(p0, HP), :],
                                      pbuf, sem).wait()

                def _grp(g, _, d=d, l=l, pbuf=pbuf):
                    gg = d * (HP // 16) + g
                    for j in range(16):
                        mo = idxs[gg * 16 + j]
                        ab = l * M * CP + mo
                        pr = g * 16 + j
                        for k in range(C // 16):
                            v = pbuf[pr, pl.ds(k * 16, 16)]
                            plsc.addupdate(acc.at[pl.ds(ab + k * 16, 16)], v)
                        v = pbuf[pr, pl.ds(CTAIL, 16)]
                        v = jnp.where(tail_keep, v, 0.0)
                        plsc.addupdate(acc.at[pl.ds(ab + CTAIL, 16)], v)
                    return 0

                lax.fori_loop(0, HP // 16, _grp, 0)

                @pl.when(l + 1 < L)
                def _next(d=d, l=l, b=b, pbuf=pbuf, sem=sem):
                    pltpu.async_copy(
                        pred.at[b, l + 1, pl.ds(p0 + d * HP, HP), :],
                        pbuf, sem)
            return 0

        lax.fori_loop(0, L, _layer, 0)

        pltpu.sync_copy(acc, sums_out.at[wid, pl.ds(b * ACC, ACC)])

    pltpu.sync_copy(cnt, cnt_out.at[wid])


def _make_sc_kernel():
    mesh = plsc.VectorSubcoreMesh(core_axis_name="c", subcore_axis_name="s")
    return pl.kernel(
        _sc_kernel_body,
        out_type=[
            jax.ShapeDtypeStruct((NW, B * ACC), jnp.float32),
            jax.ShapeDtypeStruct((NW, B * M * CP), jnp.float32),
        ],
        mesh=mesh,
        compiler_params=pltpu.CompilerParams(use_tc_tiling_on_sc=True),
        scratch_types=[
            pltpu.VMEM((M, P), jnp.float32),
            pltpu.VMEM((B * 2, 128), jnp.int32),
            pltpu.VMEM((HP, C), jnp.float32),
            pltpu.VMEM((HP, C), jnp.float32),
            pltpu.VMEM((ACC,), jnp.float32),
            pltpu.VMEM((B * M * CP,), jnp.float32),
            pltpu.SMEM((P,), jnp.int32),
            pltpu.SemaphoreType.DMA,
            pltpu.SemaphoreType.DMA,
        ],
    )


def _tc_body(sa_ref, ca_ref, sb_ref, cb_ref, o_ref):
    SA = jnp.sum(sa_ref[...], axis=0).reshape(B, L, M, CP)[..., :C]
    SB = jnp.sum(sb_ref[...], axis=0).reshape(B, L, M, CP)[..., :C]
    KA = jnp.sum(ca_ref[...], axis=0).reshape(B, M, CP)[:, :, 0:1]
    KB = jnp.sum(cb_ref[...], axis=0).reshape(B, M, CP)[:, :, 0:1]

    def means(S, Kc):
        cnt = Kc.reshape(B, 1, M, 1)
        return jnp.where(cnt > 0, S / jnp.maximum(cnt, 1.0), 0.0)

    F1 = means(SA, KA)
    F2 = means(SB, KB)

    loss = jnp.zeros((L,), jnp.float32)
    nobj = jnp.zeros((), jnp.float32)
    for b in range(B):
        f1 = F1[b]
        f2 = F2[b]
        mask_obj = jnp.logical_and(jnp.sum(f1[0], axis=1) != 0,
                                   jnp.sum(f2[0], axis=1) != 0)
        maskf = mask_obj.astype(jnp.float32)
        t1 = f1 - jnp.max(f1, axis=2, keepdims=True)
        tgt = jnp.exp(t1)
        tgt = tgt / jnp.sum(tgt, axis=2, keepdims=True)
        t2 = f2 - jnp.max(f2, axis=2, keepdims=True)
        logp = t2 - jnp.log(jnp.sum(jnp.exp(t2), axis=2, keepdims=True))
        CE = -jnp.sum(tgt * logp, axis=2)
        loss = loss + jnp.sum(CE * maskf[None, :], axis=1) / jnp.maximum(
            jnp.sum(maskf), 1.0)
        nobj = nobj + jnp.sum(maskf)
    o_ref[...] = loss / jnp.maximum(nobj, 1.0)


def _tc_tail(sa, ca, sb, cb):
    return pl.pallas_call(
        _tc_body,
        out_shape=jax.ShapeDtypeStruct((L,), jnp.float32),
    )(sa, ca, sb, cb)


@jax.jit
def kernel(pred0, pred1, masks0, masks1):
    zeros_acc = jnp.zeros((ACC,), jnp.float32)
    zeros_cnt = jnp.zeros((B * M * CP,), jnp.float32)
    sck = _make_sc_kernel()
    sa, ca = sck(pred0, masks0, zeros_acc, zeros_cnt)
    sb, cb = sck(pred1, masks1, zeros_acc, zeros_cnt)
    return _tc_tail(sa, ca, sb, cb)


# speedup vs baseline: 1.8856x; 1.1336x over previous
import functools

import jax
import jax.numpy as jnp
from jax import lax
from jax.experimental import pallas as pl
from jax.experimental.pallas import tpu as pltpu
from jax.experimental.pallas import tpu_sc as plsc

B, L, N, C, M = 2, 10, 8192, 100, 32
NCORES, NSUB = 2, 16
NW = NCORES * NSUB
P = N // NW
HP = P // 2
CP = 128
ACC = L * M * CP
CTAIL = 84


def _sc_kernel_body(pred, masksf, zeros_acc, zeros_cnt,
                    sums_out, cnt_out,
                    mbuf, idx_ref, pbuf0, pbuf1, acc, cnt, idxs,
                    sem0, sem1):
    cid = lax.axis_index("c")
    sid = lax.axis_index("s")
    wid = sid * NCORES + cid
    p0 = wid * P

    pltpu.sync_copy(zeros_cnt, cnt)

    iota = jax.lax.broadcasted_iota(jnp.int32, (16,), 0)
    tail_keep = iota >= (2 * 16 - (C - CTAIL))
    ones16 = jnp.ones((16,), jnp.float32)

    for b in range(B):
        pltpu.sync_copy(masksf.at[b, :, pl.ds(p0, P)], mbuf)

        @plsc.parallel_loop(0, P // 16)
        def _group(g, b=b):
            col = g * 16
            best = mbuf[0, pl.ds(col, 16)]
            bidx = jnp.zeros((16,), jnp.int32)
            for m in range(1, M):
                v = mbuf[m, pl.ds(col, 16)]
                gt = v > best
                bidx = jnp.where(gt, jnp.full((16,), m, jnp.int32), bidx)
                best = jnp.maximum(v, best)
            idx_ref[b * 2 + g // 8, pl.ds((g % 8) * 16, 16)] = bidx

    pbufs = (pbuf0, pbuf1)
    sems = (sem0, sem1)
    for b in range(B):
        @plsc.parallel_loop(0, P // 16)
        def _grp_idx(g, b=b):
            bidx = idx_ref[b * 2 + g // 8, pl.ds((g % 8) * 16, 16)]
            for j in range(16):
                m = bidx[j]
                idxs[g * 16 + j] = m * CP
                plsc.addupdate(cnt.at[pl.ds((b * M + m) * CP, 16)], ones16)

        pltpu.sync_copy(zeros_acc, acc)

        pltpu.async_copy(pred.at[b, 0, pl.ds(p0, HP), :], pbuf0, sem0)
        pltpu.async_copy(pred.at[b, 0, pl.ds(p0 + HP, HP), :], pbuf1, sem1)

        def _layer(l, _, b=b):
            for d in range(2):
                pbuf = pbufs[d]
                sem = sems[d]
                pltpu.make_async_copy(pred.at[b, 0, pl.ds(p0, HP), :],
                                      pbuf, sem).wait()

                @plsc.parallel_loop(0, HP // 16)
                def _grp(g, d=d, l=l, pbuf=pbuf):
                    gg = d * (HP // 16) + g
                    for j in range(16):
                        mo = idxs[gg * 16 + j]
                        ab = l * M * CP + mo
                        pr = g * 16 + j
                        for k in range(C // 16):
                            v = pbuf[pr, pl.ds(k * 16, 16)]
                            plsc.addupdate(acc.at[pl.ds(ab + k * 16, 16)], v)
                        v = pbuf[pr, pl.ds(CTAIL, 16)]
                        v = jnp.where(tail_keep, v, 0.0)
                        plsc.addupdate(acc.at[pl.ds(ab + CTAIL, 16)], v)

                @pl.when(l + 1 < L)
                def _next(d=d, l=l, b=b, pbuf=pbuf, sem=sem):
                    pltpu.async_copy(
                        pred.at[b, l + 1, pl.ds(p0 + d * HP, HP), :],
                        pbuf, sem)
            return 0

        lax.fori_loop(0, L, _layer, 0)

        pltpu.sync_copy(acc, sums_out.at[wid, pl.ds(b * ACC, ACC)])

    pltpu.sync_copy(cnt, cnt_out.at[wid])


def _make_sc_kernel():
    mesh = plsc.VectorSubcoreMesh(core_axis_name="c", subcore_axis_name="s")
    return pl.kernel(
        _sc_kernel_body,
        out_type=[
            jax.ShapeDtypeStruct((NW, B * ACC), jnp.float32),
            jax.ShapeDtypeStruct((NW, B * M * CP), jnp.float32),
        ],
        mesh=mesh,
        compiler_params=pltpu.CompilerParams(use_tc_tiling_on_sc=True),
        scratch_types=[
            pltpu.VMEM((M, P), jnp.float32),
            pltpu.VMEM((B * 2, 128), jnp.int32),
            pltpu.VMEM((HP, C), jnp.float32),
            pltpu.VMEM((HP, C), jnp.float32),
            pltpu.VMEM((ACC,), jnp.float32),
            pltpu.VMEM((B * M * CP,), jnp.float32),
            pltpu.SMEM((P,), jnp.int32),
            pltpu.SemaphoreType.DMA,
            pltpu.SemaphoreType.DMA,
        ],
    )


def _tc_body(sa_ref, ca_ref, sb_ref, cb_ref, o_ref):
    SA = jnp.sum(sa_ref[...], axis=0).reshape(B, L, M, CP)[..., :C]
    SB = jnp.sum(sb_ref[...], axis=0).reshape(B, L, M, CP)[..., :C]
    KA = jnp.sum(ca_ref[...], axis=0).reshape(B, M, CP)[:, :, 0:1]
    KB = jnp.sum(cb_ref[...], axis=0).reshape(B, M, CP)[:, :, 0:1]

    def means(S, Kc):
        cnt = Kc.reshape(B, 1, M, 1)
        return jnp.where(cnt > 0, S / jnp.maximum(cnt, 1.0), 0.0)

    F1 = means(SA, KA)
    F2 = means(SB, KB)

    loss = jnp.zeros((L,), jnp.float32)
    nobj = jnp.zeros((), jnp.float32)
    for b in range(B):
        f1 = F1[b]
        f2 = F2[b]
        mask_obj = jnp.logical_and(jnp.sum(f1[0], axis=1) != 0,
                                   jnp.sum(f2[0], axis=1) != 0)
        maskf = mask_obj.astype(jnp.float32)
        t1 = f1 - jnp.max(f1, axis=2, keepdims=True)
        tgt = jnp.exp(t1)
        tgt = tgt / jnp.sum(tgt, axis=2, keepdims=True)
        t2 = f2 - jnp.max(f2, axis=2, keepdims=True)
        logp = t2 - jnp.log(jnp.sum(jnp.exp(t2), axis=2, keepdims=True))
        CE = -jnp.sum(tgt * logp, axis=2)
        loss = loss + jnp.sum(CE * maskf[None, :], axis=1) / jnp.maximum(
            jnp.sum(maskf), 1.0)
        nobj = nobj + jnp.sum(maskf)
    o_ref[...] = loss / jnp.maximum(nobj, 1.0)


def _tc_tail(sa, ca, sb, cb):
    return pl.pallas_call(
        _tc_body,
        out_shape=jax.ShapeDtypeStruct((L,), jnp.float32),
    )(sa, ca, sb, cb)


@jax.jit
def kernel(pred0, pred1, masks0, masks1):
    zeros_acc = jnp.zeros((ACC,), jnp.float32)
    zeros_cnt = jnp.zeros((B * M * CP,), jnp.float32)
    sck = _make_sc_kernel()
    sa, ca = sck(pred0, masks0, zeros_acc, zeros_cnt)
    sb, cb = sck(pred1, masks1, zeros_acc, zeros_cnt)
    return _tc_tail(sa, ca, sb, cb)


# speedup vs baseline: 3.4018x; 1.8041x over previous
import functools

import jax
import jax.numpy as jnp
from jax import lax
from jax.experimental import pallas as pl
from jax.experimental.pallas import tpu as pltpu
from jax.experimental.pallas import tpu_sc as plsc

B, L, N, C, M = 2, 10, 8192, 100, 32
NCORES, NSUB = 2, 16
NW = NCORES * NSUB
P = N // NW
HP = P // 2
CP = 128
ACC = L * M * CP
CTAIL = 84


def _sc_kernel_body(pred, masksf, zeros_acc, zeros_cnt,
                    sums_out, cnt_out,
                    mbuf, idx_ref, pbuf0, pbuf1, acc, cnt, idxs,
                    sem0, sem1):
    cid = lax.axis_index("c")
    sid = lax.axis_index("s")
    wid = sid * NCORES + cid
    p0 = wid * P

    pltpu.sync_copy(zeros_cnt, cnt)

    iota = jax.lax.broadcasted_iota(jnp.int32, (16,), 0)
    tail_keep = iota >= (2 * 16 - (C - CTAIL))
    ones16 = jnp.ones((16,), jnp.float32)

    for b in range(B):
        pltpu.sync_copy(masksf.at[b, :, pl.ds(p0, P)], mbuf)

        @plsc.parallel_loop(0, P // 16)
        def _group(g, b=b):
            col = g * 16
            best = mbuf[0, pl.ds(col, 16)]
            bidx = jnp.zeros((16,), jnp.int32)
            for m in range(1, M):
                v = mbuf[m, pl.ds(col, 16)]
                gt = v > best
                bidx = jnp.where(gt, jnp.full((16,), m, jnp.int32), bidx)
                best = jnp.maximum(v, best)
            idx_ref[b * 2 + g // 8, pl.ds((g % 8) * 16, 16)] = bidx

    pbufs = (pbuf0, pbuf1)
    sems = (sem0, sem1)
    for b in range(B):
        @plsc.parallel_loop(0, P // 16)
        def _grp_idx(g, b=b):
            bidx = idx_ref[b * 2 + g // 8, pl.ds((g % 8) * 16, 16)]
            for j in range(16):
                m = bidx[j]
                idxs[g * 16 + j] = m * CP
                plsc.addupdate(cnt.at[pl.ds((b * M + m) * CP, 16)], ones16)

        pltpu.sync_copy(zeros_acc, acc)

        pltpu.async_copy(pred.at[b, 0, pl.ds(p0, HP), :], pbuf0, sem0)
        pltpu.async_copy(pred.at[b, 0, pl.ds(p0 + HP, HP), :], pbuf1, sem1)

        def _layer(l, _, b=b):
            for d in range(2):
                pbuf = pbufs[d]
                sem = sems[d]
                pltpu.make_async_copy(pred.at[b, 0, pl.ds(p0, HP), :],
                                      pbuf, sem).wait()

                @plsc.parallel_loop(0, HP // 16, unroll=2)
                def _grp(g, d=d, l=l, pbuf=pbuf):
                    gg = d * (HP // 16) + g
                    for j in range(16):
                        mo = idxs[gg * 16 + j]
                        ab = l * M * CP + mo
                        pr = g * 16 + j
                        for k in range(C // 16):
                            v = pbuf[pr, pl.ds(k * 16, 16)]
                            plsc.addupdate(acc.at[pl.ds(ab + k * 16, 16)], v)
                        v = pbuf[pr, pl.ds(CTAIL, 16)]
                        v = jnp.where(tail_keep, v, 0.0)
                        plsc.addupdate(acc.at[pl.ds(ab + CTAIL, 16)], v)

                @pl.when(l + 1 < L)
                def _next(d=d, l=l, b=b, pbuf=pbuf, sem=sem):
                    pltpu.async_copy(
                        pred.at[b, l + 1, pl.ds(p0 + d * HP, HP), :],
                        pbuf, sem)
            return 0

        lax.fori_loop(0, L, _layer, 0)

        pltpu.sync_copy(acc, sums_out.at[wid, pl.ds(b * ACC, ACC)])

    pltpu.sync_copy(cnt, cnt_out.at[wid])


def _make_sc_kernel():
    mesh = plsc.VectorSubcoreMesh(core_axis_name="c", subcore_axis_name="s")
    return pl.kernel(
        _sc_kernel_body,
        out_type=[
            jax.ShapeDtypeStruct((NW, B * ACC), jnp.float32),
            jax.ShapeDtypeStruct((NW, B * M * CP), jnp.float32),
        ],
        mesh=mesh,
        compiler_params=pltpu.CompilerParams(use_tc_tiling_on_sc=True),
        scratch_types=[
            pltpu.VMEM((M, P), jnp.float32),
            pltpu.VMEM((B * 2, 128), jnp.int32),
            pltpu.VMEM((HP, C), jnp.float32),
            pltpu.VMEM((HP, C), jnp.float32),
            pltpu.VMEM((ACC,), jnp.float32),
            pltpu.VMEM((B * M * CP,), jnp.float32),
            pltpu.SMEM((P,), jnp.int32),
            pltpu.SemaphoreType.DMA,
            pltpu.SemaphoreType.DMA,
        ],
    )


def _tc_body(sa_ref, ca_ref, sb_ref, cb_ref, o_ref):
    SA = jnp.sum(sa_ref[...], axis=0).reshape(B, L, M, CP)[..., :C]
    SB = jnp.sum(sb_ref[...], axis=0).reshape(B, L, M, CP)[..., :C]
    KA = jnp.sum(ca_ref[...], axis=0).reshape(B, M, CP)[:, :, 0:1]
    KB = jnp.sum(cb_ref[...], axis=0).reshape(B, M, CP)[:, :, 0:1]

    def means(S, Kc):
        cnt = Kc.reshape(B, 1, M, 1)
        return jnp.where(cnt > 0, S / jnp.maximum(cnt, 1.0), 0.0)

    F1 = means(SA, KA)
    F2 = means(SB, KB)

    loss = jnp.zeros((L,), jnp.float32)
    nobj = jnp.zeros((), jnp.float32)
    for b in range(B):
        f1 = F1[b]
        f2 = F2[b]
        mask_obj = jnp.logical_and(jnp.sum(f1[0], axis=1) != 0,
                                   jnp.sum(f2[0], axis=1) != 0)
        maskf = mask_obj.astype(jnp.float32)
        t1 = f1 - jnp.max(f1, axis=2, keepdims=True)
        tgt = jnp.exp(t1)
        tgt = tgt / jnp.sum(tgt, axis=2, keepdims=True)
        t2 = f2 - jnp.max(f2, axis=2, keepdims=True)
        logp = t2 - jnp.log(jnp.sum(jnp.exp(t2), axis=2, keepdims=True))
        CE = -jnp.sum(tgt * logp, axis=2)
        loss = loss + jnp.sum(CE * maskf[None, :], axis=1) / jnp.maximum(
            jnp.sum(maskf), 1.0)
        nobj = nobj + jnp.sum(maskf)
    o_ref[...] = loss / jnp.maximum(nobj, 1.0)


def _tc_tail(sa, ca, sb, cb):
    return pl.pallas_call(
        _tc_body,
        out_shape=jax.ShapeDtypeStruct((L,), jnp.float32),
    )(sa, ca, sb, cb)


@jax.jit
def kernel(pred0, pred1, masks0, masks1):
    zeros_acc = jnp.zeros((ACC,), jnp.float32)
    zeros_cnt = jnp.zeros((B * M * CP,), jnp.float32)
    sck = _make_sc_kernel()
    sa, ca = sck(pred0, masks0, zeros_acc, zeros_cnt)
    sb, cb = sck(pred1, masks1, zeros_acc, zeros_cnt)
    return _tc_tail(sa, ca, sb, cb)
